# baseline (speedup 1.0000x reference)
import functools

import jax
import jax.numpy as jnp
from jax import lax
from jax.experimental import pallas as pl
from jax.experimental.pallas import tpu as pltpu
from jax.experimental.pallas import tpu_sc as plsc

N_NODES = 10000
N_EDGES = 320000
D_IN = 128
D_HID = 128
N_CLS = 16

NC = 2
NS = 16
NP = NS * 640
RPT = NP // NS

CHUNK1 = 64
CH1_C0 = 188
CH1_C1 = 128
EP1 = NS * (CH1_C0 + CH1_C1) * CHUNK1

CHUNK2 = 128
CH2_C0 = 92
CH2_C1 = 68
EP2 = NS * (CH2_C0 + CH2_C1) * CHUNK2

EFLAT = 330752
M1 = EFLAT // CHUNK1
M2 = EFLAT // CHUNK2


def _make_agg(d, with_cnt, chunk, ch0, ch1):
  ch_max = max(ch0, ch1)
  assert ch0 % 4 == 0 and ch1 % 4 == 0
  out_type = [jax.ShapeDtypeStruct((NC, NP, d), jnp.float32)]
  if with_cnt:
    out_type.append(jax.ShapeDtypeStruct((NC, NP), jnp.float32))

  scratch = [
      pltpu.VMEM((4, chunk), jnp.int32),
      pltpu.VMEM((ch_max, chunk), jnp.int32),
      [pltpu.VMEM((chunk, d), jnp.float32)] * 4,
      pltpu.VMEM_SHARED((NP, d), jnp.float32),
      [pltpu.SemaphoreType.DMA] * 4,
      [pltpu.SemaphoreType.DMA] * 4,
      [pltpu.SemaphoreType.DMA] * 4,
  ]
  if with_cnt:
    scratch += [
        pltpu.VMEM((chunk,), jnp.float32),
        pltpu.VMEM((RPT,), jnp.float32),
        pltpu.VMEM_SHARED((NP,), jnp.float32),
        pltpu.SemaphoreType.DMA,
    ]

  mesh = plsc.VectorSubcoreMesh(core_axis_name="c", subcore_axis_name="s")

  @functools.partial(pl.kernel, mesh=mesh, out_type=out_type,
                     scratch_types=scratch,
                     compiler_params=pltpu.CompilerParams(
                         use_tc_tiling_on_sc=False))
  def body(feat_hbm, src_hbm, dst_hbm, *rest):
    if with_cnt:
      (agg_out, cnt_out, srr_v, dst_v, rows, agg_sh, isem, gsem, ssem,
       ones_v, zc_v, cnt_sh, csem) = rest
    else:
      (agg_out, srr_v, dst_v, rows, agg_sh, isem, gsem, ssem) = rest

    cid = lax.axis_index("c")
    sid = lax.axis_index("s")
    n_my = jnp.where(cid == 0, ch0, ch1)
    row0 = jnp.where(cid == 0, sid * ch0, NS * ch0 + sid * ch1)

    pltpu.sync_copy(dst_hbm.at[pl.ds(row0, ch_max)], dst_v)

    z16 = jnp.zeros((16,), jnp.float32)
    g = d // 16

    def zrow(i, c):
      rows[0][i // g, pl.ds((i % g) * 16, 16)] = z16
      return c
    lax.fori_loop(0, chunk * g, zrow, 0)
    full, rem = divmod(RPT, chunk)
    for k in range(full):
      pltpu.sync_copy(rows[0],
                      agg_sh.at[pl.ds(sid * RPT + k * chunk, chunk)])
    if rem:
      pltpu.sync_copy(rows[0].at[pl.ds(0, rem)],
                      agg_sh.at[pl.ds(sid * RPT + full * chunk, rem)])

    if with_cnt:
      one16 = jnp.ones((16,), jnp.float32)
      for k in range(chunk // 16):
        ones_v[pl.ds(k * 16, 16)] = one16

      def zcnt(i, c):
        zc_v[pl.ds(i * 16, 16)] = z16
        return c
      lax.fori_loop(0, RPT // 16, zcnt, 0)
      pltpu.sync_copy(zc_v, cnt_sh.at[pl.ds(sid * RPT, RPT)])

    plsc.subcore_barrier()

    def fire_src(jj, sl):
      pltpu.async_copy(src_hbm.at[pl.ds(row0 + jj, 1)],
                       srr_v.at[pl.ds(sl, 1)], isem[sl])

    def wait_src(sl):
      pltpu.make_async_copy(src_hbm.at[pl.ds(row0, 1)],
                            srr_v.at[pl.ds(sl, 1)], isem[sl]).wait()

    def fire_gather(b):
      pltpu.async_copy(feat_hbm.at[srr_v.at[b]], rows[b], gsem[b])

    def wait_gather(b):
      pltpu.make_async_copy(feat_hbm.at[srr_v.at[b]], rows[b],
                            gsem[b]).wait()

    def wait_scatter(b):
      pltpu.make_async_copy(rows[b], agg_sh.at[dst_v.at[0]],
                            ssem[b]).wait()

    def step(j, b, guarded):
      sl2 = (b + 2) % 4
      wait_src(sl2)
      wait_gather(b)
      pltpu.async_copy(rows[b], agg_sh.at[dst_v.at[j]], ssem[b], add=True)
      if with_cnt:
        pltpu.async_copy(ones_v, cnt_sh.at[dst_v.at[j]], csem, add=True)
      if not guarded:
        wait_scatter(sl2)
      fire_gather(sl2)
      fire_src(lax.rem(j + 4, n_my), b)

    for k in range(4):
      fire_src(jnp.int32(k), k)
    for k in range(2):
      wait_src(k)
      fire_gather(k)

    for b in range(4):
      step(jnp.int32(b), b, guarded=b < 2)

    def ring_pass(j4, c):
      for b in range(4):
        step(4 * j4 + b, b, guarded=False)
      return c
    lax.fori_loop(1, n_my // 4, ring_pass, 0)

    wait_gather(0)
    wait_gather(1)
    wait_scatter(2)
    wait_scatter(3)
    wait_src(2)
    wait_src(3)
    if with_cnt:
      def cnt_drain(j, c):
        pltpu.make_async_copy(ones_v, cnt_sh.at[dst_v.at[0]], csem).wait()
        return c
      lax.fori_loop(0, n_my, cnt_drain, 0)

    plsc.subcore_barrier()

    r0 = sid * RPT
    pltpu.sync_copy(agg_sh.at[pl.ds(r0, RPT)],
                    agg_out.at[cid, pl.ds(r0, RPT)])
    if with_cnt:
      pltpu.sync_copy(cnt_sh.at[pl.ds(r0, RPT)],
                      cnt_out.at[cid, pl.ds(r0, RPT)])

  return body


_agg_l1 = _make_agg(D_IN, True, CHUNK1, CH1_C0, CH1_C1)
_agg_l2 = _make_agg(N_CLS, False, CHUNK2, CH2_C0, CH2_C1)

BLK = 1024
GRID = NP // BLK


def _tc1_body(aggp, cnt_t, xp, w1l, w1r, b1, w2l, w2r, b2,
              h_out, y2_out, z2_out):
  agg = aggp[0] + aggp[1]
  cnt = cnt_t[:, 0] + cnt_t[:, 1]
  inv = 1.0 / jnp.maximum(cnt, 1.0)
  mean = agg * inv[:, None]
  h = mean @ w1l[...] + xp[...] @ w1r[...] + b1[...]
  h = jnp.maximum(h, 0.0)
  h_out[...] = h
  y2_out[...] = h @ w2l[...]
  z2_out[...] = h @ w2r[...] + b2[...]


_tc1 = pl.pallas_call(
    _tc1_body,
    grid=(GRID,),
    in_specs=[
        pl.BlockSpec((NC, BLK, D_IN), lambda i: (0, i, 0)),
        pl.BlockSpec((BLK, NC), lambda i: (i, 0)),
        pl.BlockSpec((BLK, D_IN), lambda i: (i, 0)),
        pl.BlockSpec((D_IN, D_HID), lambda i: (0, 0)),
        pl.BlockSpec((D_IN, D_HID), lambda i: (0, 0)),
        pl.BlockSpec((1, D_HID), lambda i: (0, 0)),
        pl.BlockSpec((D_HID, N_CLS), lambda i: (0, 0)),
        pl.BlockSpec((D_HID, N_CLS), lambda i: (0, 0)),
        pl.BlockSpec((1, N_CLS), lambda i: (0, 0)),
    ],
    out_specs=[
        pl.BlockSpec((BLK, D_HID), lambda i: (i, 0)),
        pl.BlockSpec((BLK, N_CLS), lambda i: (i, 0)),
        pl.BlockSpec((BLK, N_CLS), lambda i: (i, 0)),
    ],
    out_shape=[
        jax.ShapeDtypeStruct((NP, D_HID), jnp.float32),
        jax.ShapeDtypeStruct((NP, N_CLS), jnp.float32),
        jax.ShapeDtypeStruct((NP, N_CLS), jnp.float32),
    ],
)


def _tc2_body(agg2p, cnt_t, z2, out):
  s = agg2p[0] + agg2p[1]
  cnt = cnt_t[:, 0] + cnt_t[:, 1]
  inv = 1.0 / jnp.maximum(cnt, 1.0)
  out[...] = (s * inv[:, None] + z2[...])[:N_NODES]


_tc2 = pl.pallas_call(
    _tc2_body,
    in_specs=[
        pl.BlockSpec((NC, NP, N_CLS), lambda: (0, 0, 0)),
        pl.BlockSpec((NP, NC), lambda: (0, 0)),
        pl.BlockSpec((NP, N_CLS), lambda: (0, 0)),
    ],
    out_specs=pl.BlockSpec((N_NODES, N_CLS), lambda: (0, 0)),
    out_shape=jax.ShapeDtypeStruct((N_NODES, N_CLS), jnp.float32),
)


def kernel(x, edge_index, W1_l, W1_r, b1, W2_l, W2_r, b2):
  src = edge_index[0].astype(jnp.int32)
  dst = edge_index[1].astype(jnp.int32)
  pad = EFLAT - N_EDGES
  srcp = jnp.concatenate([src, jnp.zeros((pad,), jnp.int32)])
  dstp = jnp.concatenate([dst, jnp.full((pad,), NP - 1, jnp.int32)])

  aggp, cntp = _agg_l1(x, srcp.reshape(M1, CHUNK1), dstp.reshape(M1, CHUNK1))
  cnt_t = cntp.T
  xp = jnp.pad(x, ((0, NP - N_NODES), (0, 0)))

  h, y2, z2 = _tc1(aggp, cnt_t, xp, W1_l, W1_r, b1.reshape(1, -1),
                   W2_l, W2_r, b2.reshape(1, -1))

  (agg2p,) = _agg_l2(y2, srcp.reshape(M2, CHUNK2), dstp.reshape(M2, CHUNK2))
  return _tc2(agg2p, cnt_t, z2)


# speedup vs baseline: 1.0547x; 1.0547x over previous
import functools

import jax
import jax.numpy as jnp
from jax import lax
from jax.experimental import pallas as pl
from jax.experimental.pallas import tpu as pltpu
from jax.experimental.pallas import tpu_sc as plsc

N_NODES = 10000
N_EDGES = 320000
D_IN = 128
D_HID = 128
N_CLS = 16

NC = 2
NS = 16
NP = NS * 640
RPT = NP // NS

CHUNK1 = 64
CH1_C0 = 228
CH1_C1 = 88
EP1 = NS * (CH1_C0 + CH1_C1) * CHUNK1

CHUNK2 = 128
CH2_C0 = 100
CH2_C1 = 60
EP2 = NS * (CH2_C0 + CH2_C1) * CHUNK2

EFLAT = 332800
M1 = EFLAT // CHUNK1
M2 = EFLAT // CHUNK2


def _make_agg(d, with_cnt, chunk, ch0, ch1):
  ch_max = max(ch0, ch1)
  assert ch0 % 4 == 0 and ch1 % 4 == 0
  out_type = [jax.ShapeDtypeStruct((NC, NP, d), jnp.float32)]
  if with_cnt:
    out_type.append(jax.ShapeDtypeStruct((NC, NP), jnp.float32))

  scratch = [
      pltpu.VMEM((4, chunk), jnp.int32),
      pltpu.VMEM((ch_max, chunk), jnp.int32),
      [pltpu.VMEM((chunk, d), jnp.float32)] * 4,
      pltpu.VMEM_SHARED((NP, d), jnp.float32),
      [pltpu.SemaphoreType.DMA] * 4,
      [pltpu.SemaphoreType.DMA] * 4,
      [pltpu.SemaphoreType.DMA] * 4,
  ]
  if with_cnt:
    scratch += [
        pltpu.VMEM((chunk,), jnp.float32),
        pltpu.VMEM((RPT,), jnp.float32),
        pltpu.VMEM_SHARED((NP,), jnp.float32),
        pltpu.SemaphoreType.DMA,
    ]

  mesh = plsc.VectorSubcoreMesh(core_axis_name="c", subcore_axis_name="s")

  @functools.partial(pl.kernel, mesh=mesh, out_type=out_type,
                     scratch_types=scratch,
                     compiler_params=pltpu.CompilerParams(
                         use_tc_tiling_on_sc=False))
  def body(feat_hbm, src_hbm, dst_hbm, *rest):
    if with_cnt:
      (agg_out, cnt_out, srr_v, dst_v, rows, agg_sh, isem, gsem, ssem,
       ones_v, zc_v, cnt_sh, csem) = rest
    else:
      (agg_out, srr_v, dst_v, rows, agg_sh, isem, gsem, ssem) = rest

    cid = lax.axis_index("c")
    sid = lax.axis_index("s")
    n_my = jnp.where(cid == 0, ch0, ch1)
    row0 = jnp.where(cid == 0, sid * ch0, NS * ch0 + sid * ch1)

    pltpu.sync_copy(dst_hbm.at[pl.ds(row0, ch_max)], dst_v)

    z16 = jnp.zeros((16,), jnp.float32)
    g = d // 16

    def zrow(i, c):
      rows[0][i // g, pl.ds((i % g) * 16, 16)] = z16
      return c
    lax.fori_loop(0, chunk * g, zrow, 0)
    full, rem = divmod(RPT, chunk)
    for k in range(full):
      pltpu.sync_copy(rows[0],
                      agg_sh.at[pl.ds(sid * RPT + k * chunk, chunk)])
    if rem:
      pltpu.sync_copy(rows[0].at[pl.ds(0, rem)],
                      agg_sh.at[pl.ds(sid * RPT + full * chunk, rem)])

    if with_cnt:
      one16 = jnp.ones((16,), jnp.float32)
      for k in range(chunk // 16):
        ones_v[pl.ds(k * 16, 16)] = one16

      def zcnt(i, c):
        zc_v[pl.ds(i * 16, 16)] = z16
        return c
      lax.fori_loop(0, RPT // 16, zcnt, 0)
      pltpu.sync_copy(zc_v, cnt_sh.at[pl.ds(sid * RPT, RPT)])

    plsc.subcore_barrier()

    def fire_src(jj, sl):
      pltpu.async_copy(src_hbm.at[pl.ds(row0 + jj, 1)],
                       srr_v.at[pl.ds(sl, 1)], isem[sl])

    def wait_src(sl):
      pltpu.make_async_copy(src_hbm.at[pl.ds(row0, 1)],
                            srr_v.at[pl.ds(sl, 1)], isem[sl]).wait()

    def fire_gather(b):
      pltpu.async_copy(feat_hbm.at[srr_v.at[b]], rows[b], gsem[b])

    def wait_gather(b):
      pltpu.make_async_copy(feat_hbm.at[srr_v.at[b]], rows[b],
                            gsem[b]).wait()

    def wait_scatter(b):
      pltpu.make_async_copy(rows[b], agg_sh.at[dst_v.at[0]],
                            ssem[b]).wait()

    def step(j, b, guarded):
      sl2 = (b + 2) % 4
      wait_src(sl2)
      wait_gather(b)
      pltpu.async_copy(rows[b], agg_sh.at[dst_v.at[j]], ssem[b], add=True)
      if with_cnt:
        pltpu.async_copy(ones_v, cnt_sh.at[dst_v.at[j]], csem, add=True)
      if not guarded:
        wait_scatter(sl2)
      fire_gather(sl2)
      fire_src(lax.rem(j + 4, n_my), b)

    for k in range(4):
      fire_src(jnp.int32(k), k)
    for k in range(2):
      wait_src(k)
      fire_gather(k)

    for b in range(4):
      step(jnp.int32(b), b, guarded=b < 2)

    def ring_pass(j4, c):
      for b in range(4):
        step(4 * j4 + b, b, guarded=False)
      return c
    lax.fori_loop(1, n_my // 4, ring_pass, 0)

    wait_gather(0)
    wait_gather(1)
    wait_scatter(2)
    wait_scatter(3)
    wait_src(2)
    wait_src(3)
    if with_cnt:
      def cnt_drain(j, c):
        pltpu.make_async_copy(ones_v, cnt_sh.at[dst_v.at[0]], csem).wait()
        return c
      lax.fori_loop(0, n_my, cnt_drain, 0)

    plsc.subcore_barrier()

    r0 = sid * RPT
    pltpu.sync_copy(agg_sh.at[pl.ds(r0, RPT)],
                    agg_out.at[cid, pl.ds(r0, RPT)])
    if with_cnt:
      pltpu.sync_copy(cnt_sh.at[pl.ds(r0, RPT)],
                      cnt_out.at[cid, pl.ds(r0, RPT)])

  return body


_agg_l1 = _make_agg(D_IN, True, CHUNK1, CH1_C0, CH1_C1)
_agg_l2 = _make_agg(N_CLS, False, CHUNK2, CH2_C0, CH2_C1)

BLK = 1024
GRID = NP // BLK


def _tc1_body(aggp, cnt_t, xp, w1l, w1r, b1, w2l, w2r, b2,
              h_out, y2_out, z2_out):
  agg = aggp[0] + aggp[1]
  cnt = cnt_t[:, 0] + cnt_t[:, 1]
  inv = 1.0 / jnp.maximum(cnt, 1.0)
  mean = agg * inv[:, None]
  h = mean @ w1l[...] + xp[...] @ w1r[...] + b1[...]
  h = jnp.maximum(h, 0.0)
  h_out[...] = h
  y2_out[...] = h @ w2l[...]
  z2_out[...] = h @ w2r[...] + b2[...]


_tc1 = pl.pallas_call(
    _tc1_body,
    grid=(GRID,),
    in_specs=[
        pl.BlockSpec((NC, BLK, D_IN), lambda i: (0, i, 0)),
        pl.BlockSpec((BLK, NC), lambda i: (i, 0)),
        pl.BlockSpec((BLK, D_IN), lambda i: (i, 0)),
        pl.BlockSpec((D_IN, D_HID), lambda i: (0, 0)),
        pl.BlockSpec((D_IN, D_HID), lambda i: (0, 0)),
        pl.BlockSpec((1, D_HID), lambda i: (0, 0)),
        pl.BlockSpec((D_HID, N_CLS), lambda i: (0, 0)),
        pl.BlockSpec((D_HID, N_CLS), lambda i: (0, 0)),
        pl.BlockSpec((1, N_CLS), lambda i: (0, 0)),
    ],
    out_specs=[
        pl.BlockSpec((BLK, D_HID), lambda i: (i, 0)),
        pl.BlockSpec((BLK, N_CLS), lambda i: (i, 0)),
        pl.BlockSpec((BLK, N_CLS), lambda i: (i, 0)),
    ],
    out_shape=[
        jax.ShapeDtypeStruct((NP, D_HID), jnp.float32),
        jax.ShapeDtypeStruct((NP, N_CLS), jnp.float32),
        jax.ShapeDtypeStruct((NP, N_CLS), jnp.float32),
    ],
)


def _tc2_body(agg2p, cnt_t, z2, out):
  s = agg2p[0] + agg2p[1]
  cnt = cnt_t[:, 0] + cnt_t[:, 1]
  inv = 1.0 / jnp.maximum(cnt, 1.0)
  out[...] = (s * inv[:, None] + z2[...])[:N_NODES]


_tc2 = pl.pallas_call(
    _tc2_body,
    in_specs=[
        pl.BlockSpec((NC, NP, N_CLS), lambda: (0, 0, 0)),
        pl.BlockSpec((NP, NC), lambda: (0, 0)),
        pl.BlockSpec((NP, N_CLS), lambda: (0, 0)),
    ],
    out_specs=pl.BlockSpec((N_NODES, N_CLS), lambda: (0, 0)),
    out_shape=jax.ShapeDtypeStruct((N_NODES, N_CLS), jnp.float32),
)


def kernel(x, edge_index, W1_l, W1_r, b1, W2_l, W2_r, b2):
  src = edge_index[0].astype(jnp.int32)
  dst = edge_index[1].astype(jnp.int32)
  pad = EFLAT - N_EDGES
  srcp = jnp.concatenate([src, jnp.zeros((pad,), jnp.int32)])
  dstp = jnp.concatenate([dst, jnp.full((pad,), NP - 1, jnp.int32)])

  aggp, cntp = _agg_l1(x, srcp.reshape(M1, CHUNK1), dstp.reshape(M1, CHUNK1))
  cnt_t = cntp.T
  xp = jnp.pad(x, ((0, NP - N_NODES), (0, 0)))

  h, y2, z2 = _tc1(aggp, cnt_t, xp, W1_l, W1_r, b1.reshape(1, -1),
                   W2_l, W2_r, b2.reshape(1, -1))

  (agg2p,) = _agg_l2(y2, srcp.reshape(M2, CHUNK2), dstp.reshape(M2, CHUNK2))
  return _tc2(agg2p, cnt_t, z2)


# speedup vs baseline: 1.1494x; 1.0898x over previous
import functools

import jax
import jax.numpy as jnp
from jax import lax
from jax.experimental import pallas as pl
from jax.experimental.pallas import tpu as pltpu
from jax.experimental.pallas import tpu_sc as plsc

N_NODES = 10000
N_EDGES = 320000
D_IN = 128
D_HID = 128
N_CLS = 16

NC = 2
NS = 16
NP = NS * 640
RPT = NP // NS

CHUNK1 = 48
CH1_C0 = 328
CH1_C1 = 92
EP1 = NS * (CH1_C0 + CH1_C1) * CHUNK1

CHUNK2 = 128
CH2_C0 = 104
CH2_C1 = 56
EP2 = NS * (CH2_C0 + CH2_C1) * CHUNK2

EFLAT = 334080
M1 = EFLAT // CHUNK1
M2 = EFLAT // CHUNK2


def _make_agg(d, with_cnt, chunk, ch0, ch1):
  ch_max = max(ch0, ch1)
  assert ch0 % 4 == 0 and ch1 % 4 == 0 and chunk % 16 == 0
  out_type = [jax.ShapeDtypeStruct((NC, NP, d), jnp.float32)]
  if with_cnt:
    out_type.append(jax.ShapeDtypeStruct((NC, NP), jnp.float32))

  scratch = [
      pltpu.VMEM((4, chunk), jnp.int32),
      pltpu.VMEM((ch_max, chunk), jnp.int32),
      [pltpu.VMEM((chunk, d), jnp.float32)] * 4,
      pltpu.VMEM_SHARED((NP, d), jnp.float32),
      [pltpu.SemaphoreType.DMA] * 4,
      [pltpu.SemaphoreType.DMA] * 4,
      [pltpu.SemaphoreType.DMA] * 4,
  ]
  if with_cnt:
    scratch += [
        pltpu.VMEM((chunk,), jnp.float32),
        pltpu.VMEM((RPT,), jnp.float32),
        pltpu.VMEM_SHARED((NP,), jnp.float32),
        pltpu.SemaphoreType.DMA,
    ]

  mesh = plsc.VectorSubcoreMesh(core_axis_name="c", subcore_axis_name="s")

  @functools.partial(pl.kernel, mesh=mesh, out_type=out_type,
                     scratch_types=scratch,
                     compiler_params=pltpu.CompilerParams(
                         use_tc_tiling_on_sc=False))
  def body(feat_hbm, src_hbm, dst_hbm, *rest):
    if with_cnt:
      (agg_out, cnt_out, srr_v, dst_v, rows, agg_sh, isem, gsem, ssem,
       ones_v, zc_v, cnt_sh, csem) = rest
    else:
      (agg_out, srr_v, dst_v, rows, agg_sh, isem, gsem, ssem) = rest

    cid = lax.axis_index("c")
    sid = lax.axis_index("s")
    n_my = jnp.where(cid == 0, ch0, ch1)
    row0 = jnp.where(cid == 0, sid * ch0, NS * ch0 + sid * ch1)

    pltpu.sync_copy(dst_hbm.at[pl.ds(row0, ch_max)], dst_v)

    z16 = jnp.zeros((16,), jnp.float32)
    g = d // 16

    def zrow(i, c):
      rows[0][i // g, pl.ds((i % g) * 16, 16)] = z16
      return c
    lax.fori_loop(0, chunk * g, zrow, 0)
    full, rem = divmod(RPT, chunk)
    for k in range(full):
      pltpu.sync_copy(rows[0],
                      agg_sh.at[pl.ds(sid * RPT + k * chunk, chunk)])
    if rem:
      pltpu.sync_copy(rows[0].at[pl.ds(0, rem)],
                      agg_sh.at[pl.ds(sid * RPT + full * chunk, rem)])

    if with_cnt:
      one16 = jnp.ones((16,), jnp.float32)
      for k in range(chunk // 16):
        ones_v[pl.ds(k * 16, 16)] = one16

      def zcnt(i, c):
        zc_v[pl.ds(i * 16, 16)] = z16
        return c
      lax.fori_loop(0, RPT // 16, zcnt, 0)
      pltpu.sync_copy(zc_v, cnt_sh.at[pl.ds(sid * RPT, RPT)])

    plsc.subcore_barrier()

    def fire_src(jj, sl):
      pltpu.async_copy(src_hbm.at[pl.ds(row0 + jj, 1)],
                       srr_v.at[pl.ds(sl, 1)], isem[sl])

    def wait_src(sl):
      pltpu.make_async_copy(src_hbm.at[pl.ds(row0, 1)],
                            srr_v.at[pl.ds(sl, 1)], isem[sl]).wait()

    def fire_gather(b):
      pltpu.async_copy(feat_hbm.at[srr_v.at[b]], rows[b], gsem[b])

    def wait_gather(b):
      pltpu.make_async_copy(feat_hbm.at[srr_v.at[b]], rows[b],
                            gsem[b]).wait()

    def wait_scatter(b):
      pltpu.make_async_copy(rows[b], agg_sh.at[dst_v.at[0]],
                            ssem[b]).wait()

    def step(j, b, guarded):
      sl2 = (b + 2) % 4
      wait_src(sl2)
      wait_gather(b)
      pltpu.async_copy(rows[b], agg_sh.at[dst_v.at[j]], ssem[b], add=True)
      if with_cnt:
        pltpu.async_copy(ones_v, cnt_sh.at[dst_v.at[j]], csem, add=True)
      if not guarded:
        wait_scatter(sl2)
      fire_gather(sl2)
      fire_src(lax.rem(j + 4, n_my), b)

    for k in range(4):
      fire_src(jnp.int32(k), k)
    for k in range(2):
      wait_src(k)
      fire_gather(k)

    for b in range(4):
      step(jnp.int32(b), b, guarded=b < 2)

    def ring_pass(j4, c):
      for b in range(4):
        step(4 * j4 + b, b, guarded=False)
      return c
    lax.fori_loop(1, n_my // 4, ring_pass, 0)

    wait_gather(0)
    wait_gather(1)
    wait_scatter(2)
    wait_scatter(3)
    wait_src(2)
    wait_src(3)
    if with_cnt:
      def cnt_drain(j, c):
        pltpu.make_async_copy(ones_v, cnt_sh.at[dst_v.at[0]], csem).wait()
        return c
      lax.fori_loop(0, n_my, cnt_drain, 0)

    plsc.subcore_barrier()

    r0 = sid * RPT
    pltpu.sync_copy(agg_sh.at[pl.ds(r0, RPT)],
                    agg_out.at[cid, pl.ds(r0, RPT)])
    if with_cnt:
      pltpu.sync_copy(cnt_sh.at[pl.ds(r0, RPT)],
                      cnt_out.at[cid, pl.ds(r0, RPT)])

  return body


_agg_l1 = _make_agg(D_IN, True, CHUNK1, CH1_C0, CH1_C1)
_agg_l2 = _make_agg(N_CLS, False, CHUNK2, CH2_C0, CH2_C1)

BLK = 1024
GRID = NP // BLK


def _tc1_body(aggp, cnt_t, xp, w1l, w1r, b1, w2l, w2r, b2,
              h_out, y2_out, z2_out):
  agg = aggp[0] + aggp[1]
  cnt = cnt_t[:, 0] + cnt_t[:, 1]
  inv = 1.0 / jnp.maximum(cnt, 1.0)
  mean = agg * inv[:, None]
  h = mean @ w1l[...] + xp[...] @ w1r[...] + b1[...]
  h = jnp.maximum(h, 0.0)
  h_out[...] = h
  y2_out[...] = h @ w2l[...]
  z2_out[...] = h @ w2r[...] + b2[...]


_tc1 = pl.pallas_call(
    _tc1_body,
    grid=(GRID,),
    in_specs=[
        pl.BlockSpec((NC, BLK, D_IN), lambda i: (0, i, 0)),
        pl.BlockSpec((BLK, NC), lambda i: (i, 0)),
        pl.BlockSpec((BLK, D_IN), lambda i: (i, 0)),
        pl.BlockSpec((D_IN, D_HID), lambda i: (0, 0)),
        pl.BlockSpec((D_IN, D_HID), lambda i: (0, 0)),
        pl.BlockSpec((1, D_HID), lambda i: (0, 0)),
        pl.BlockSpec((D_HID, N_CLS), lambda i: (0, 0)),
        pl.BlockSpec((D_HID, N_CLS), lambda i: (0, 0)),
        pl.BlockSpec((1, N_CLS), lambda i: (0, 0)),
    ],
    out_specs=[
        pl.BlockSpec((BLK, D_HID), lambda i: (i, 0)),
        pl.BlockSpec((BLK, N_CLS), lambda i: (i, 0)),
        pl.BlockSpec((BLK, N_CLS), lambda i: (i, 0)),
    ],
    out_shape=[
        jax.ShapeDtypeStruct((NP, D_HID), jnp.float32),
        jax.ShapeDtypeStruct((NP, N_CLS), jnp.float32),
        jax.ShapeDtypeStruct((NP, N_CLS), jnp.float32),
    ],
)


def _tc2_body(agg2p, cnt_t, z2, out):
  s = agg2p[0] + agg2p[1]
  cnt = cnt_t[:, 0] + cnt_t[:, 1]
  inv = 1.0 / jnp.maximum(cnt, 1.0)
  out[...] = (s * inv[:, None] + z2[...])[:N_NODES]


_tc2 = pl.pallas_call(
    _tc2_body,
    in_specs=[
        pl.BlockSpec((NC, NP, N_CLS), lambda: (0, 0, 0)),
        pl.BlockSpec((NP, NC), lambda: (0, 0)),
        pl.BlockSpec((NP, N_CLS), lambda: (0, 0)),
    ],
    out_specs=pl.BlockSpec((N_NODES, N_CLS), lambda: (0, 0)),
    out_shape=jax.ShapeDtypeStruct((N_NODES, N_CLS), jnp.float32),
)


def kernel(x, edge_index, W1_l, W1_r, b1, W2_l, W2_r, b2):
  src = edge_index[0].astype(jnp.int32)
  dst = edge_index[1].astype(jnp.int32)
  pad = EFLAT - N_EDGES
  srcp = jnp.concatenate([src, jnp.zeros((pad,), jnp.int32)])
  dstp = jnp.concatenate([dst, jnp.full((pad,), NP - 1, jnp.int32)])

  aggp, cntp = _agg_l1(x, srcp.reshape(M1, CHUNK1), dstp.reshape(M1, CHUNK1))
  cnt_t = cntp.T
  xp = jnp.pad(x, ((0, NP - N_NODES), (0, 0)))

  h, y2, z2 = _tc1(aggp, cnt_t, xp, W1_l, W1_r, b1.reshape(1, -1),
                   W2_l, W2_r, b2.reshape(1, -1))

  (agg2p,) = _agg_l2(y2, srcp.reshape(M2, CHUNK2), dstp.reshape(M2, CHUNK2))
  return _tc2(agg2p, cnt_t, z2)


# speedup vs baseline: 1.1751x; 1.0223x over previous
import functools

import jax
import jax.numpy as jnp
from jax import lax
from jax.experimental import pallas as pl
from jax.experimental.pallas import tpu as pltpu
from jax.experimental.pallas import tpu_sc as plsc

N_NODES = 10000
N_EDGES = 320000
D_IN = 128
D_HID = 128
N_CLS = 16

NC = 2
NS = 16
NP = NS * 640
RPT = NP // NS

CHUNK1 = 48
CH1_C0 = 316
CH1_C1 = 104
EP1 = NS * (CH1_C0 + CH1_C1) * CHUNK1

CHUNK2 = 128
CH2_C0 = 104
CH2_C1 = 56
EP2 = NS * (CH2_C0 + CH2_C1) * CHUNK2

EFLAT = 334080
M1 = EFLAT // CHUNK1
M2 = EFLAT // CHUNK2


def _make_agg(d, with_cnt, chunk, ch0, ch1):
  ch_max = max(ch0, ch1)
  assert ch0 % 4 == 0 and ch1 % 4 == 0 and chunk % 16 == 0
  out_type = [jax.ShapeDtypeStruct((NC, NP, d), jnp.float32)]
  if with_cnt:
    out_type.append(jax.ShapeDtypeStruct((NC, NP), jnp.float32))

  scratch = [
      pltpu.VMEM((4, chunk), jnp.int32),
      pltpu.VMEM((ch_max, chunk), jnp.int32),
      [pltpu.VMEM((chunk, d), jnp.float32)] * 4,
      pltpu.VMEM_SHARED((NP, d), jnp.float32),
      [pltpu.SemaphoreType.DMA] * 4,
      [pltpu.SemaphoreType.DMA] * 4,
      [pltpu.SemaphoreType.DMA] * 4,
  ]
  if with_cnt:
    scratch += [
        pltpu.VMEM((chunk,), jnp.float32),
        pltpu.VMEM((RPT,), jnp.float32),
        pltpu.VMEM_SHARED((NP,), jnp.float32),
        pltpu.SemaphoreType.DMA,
    ]

  mesh = plsc.VectorSubcoreMesh(core_axis_name="c", subcore_axis_name="s")

  @functools.partial(pl.kernel, mesh=mesh, out_type=out_type,
                     scratch_types=scratch,
                     compiler_params=pltpu.CompilerParams(
                         use_tc_tiling_on_sc=False))
  def body(feat_hbm, src_hbm, dst_hbm, *rest):
    if with_cnt:
      (agg_out, cnt_out, srr_v, dst_v, rows, agg_sh, isem, gsem, ssem,
       ones_v, zc_v, cnt_sh, csem) = rest
    else:
      (agg_out, srr_v, dst_v, rows, agg_sh, isem, gsem, ssem) = rest

    cid = lax.axis_index("c")
    sid = lax.axis_index("s")
    n_my = jnp.where(cid == 0, ch0, ch1)
    row0 = jnp.where(cid == 0, sid * ch0, NS * ch0 + sid * ch1)

    pltpu.sync_copy(dst_hbm.at[pl.ds(row0, ch_max)], dst_v)

    z16 = jnp.zeros((16,), jnp.float32)
    g = d // 16

    def zrow(i, c):
      rows[0][i // g, pl.ds((i % g) * 16, 16)] = z16
      return c
    lax.fori_loop(0, chunk * g, zrow, 0)
    full, rem = divmod(RPT, chunk)
    for k in range(full):
      pltpu.sync_copy(rows[0],
                      agg_sh.at[pl.ds(sid * RPT + k * chunk, chunk)])
    if rem:
      pltpu.sync_copy(rows[0].at[pl.ds(0, rem)],
                      agg_sh.at[pl.ds(sid * RPT + full * chunk, rem)])

    if with_cnt:
      one16 = jnp.ones((16,), jnp.float32)
      for k in range(chunk // 16):
        ones_v[pl.ds(k * 16, 16)] = one16

      def zcnt(i, c):
        zc_v[pl.ds(i * 16, 16)] = z16
        return c
      lax.fori_loop(0, RPT // 16, zcnt, 0)
      pltpu.sync_copy(zc_v, cnt_sh.at[pl.ds(sid * RPT, RPT)])

    plsc.subcore_barrier()

    def fire_src(jj, sl):
      pltpu.async_copy(src_hbm.at[pl.ds(row0 + jj, 1)],
                       srr_v.at[pl.ds(sl, 1)], isem[sl])

    def wait_src(sl):
      pltpu.make_async_copy(src_hbm.at[pl.ds(row0, 1)],
                            srr_v.at[pl.ds(sl, 1)], isem[sl]).wait()

    def fire_gather(b):
      pltpu.async_copy(feat_hbm.at[srr_v.at[b]], rows[b], gsem[b])

    def wait_gather(b):
      pltpu.make_async_copy(feat_hbm.at[srr_v.at[b]], rows[b],
                            gsem[b]).wait()

    def wait_scatter(b):
      pltpu.make_async_copy(rows[b], agg_sh.at[dst_v.at[0]],
                            ssem[b]).wait()

    def step(j, b, guarded):
      sl2 = (b + 2) % 4
      wait_src(sl2)
      wait_gather(b)
      pltpu.async_copy(rows[b], agg_sh.at[dst_v.at[j]], ssem[b], add=True)
      if with_cnt:
        pltpu.async_copy(ones_v, cnt_sh.at[dst_v.at[j]], csem, add=True)
      if not guarded:
        wait_scatter(sl2)
      fire_gather(sl2)
      fire_src(lax.rem(j + 4, n_my), b)

    for k in range(4):
      fire_src(jnp.int32(k), k)
    for k in range(2):
      wait_src(k)
      fire_gather(k)

    for b in range(4):
      step(jnp.int32(b), b, guarded=b < 2)

    def ring_pass(j4, c):
      for b in range(4):
        step(4 * j4 + b, b, guarded=False)
      return c
    lax.fori_loop(1, n_my // 4, ring_pass, 0)

    wait_gather(0)
    wait_gather(1)
    wait_scatter(2)
    wait_scatter(3)
    wait_src(2)
    wait_src(3)
    if with_cnt:
      def cnt_drain(j, c):
        pltpu.make_async_copy(ones_v, cnt_sh.at[dst_v.at[0]], csem).wait()
        return c
      lax.fori_loop(0, n_my, cnt_drain, 0)

    plsc.subcore_barrier()

    r0 = sid * RPT
    pltpu.sync_copy(agg_sh.at[pl.ds(r0, RPT)],
                    agg_out.at[cid, pl.ds(r0, RPT)])
    if with_cnt:
      pltpu.sync_copy(cnt_sh.at[pl.ds(r0, RPT)],
                      cnt_out.at[cid, pl.ds(r0, RPT)])

  return body


_agg_l1 = _make_agg(D_IN, True, CHUNK1, CH1_C0, CH1_C1)
_agg_l2 = _make_agg(N_CLS, False, CHUNK2, CH2_C0, CH2_C1)

BLK = 1024
GRID = NP // BLK


def _tc1_body(aggp, cnt_t, xp, w1l, w1r, b1, w2l, w2r, b2,
              h_out, y2_out, z2_out):
  agg = aggp[0] + aggp[1]
  cnt = cnt_t[:, 0] + cnt_t[:, 1]
  inv = 1.0 / jnp.maximum(cnt, 1.0)
  mean = agg * inv[:, None]
  h = mean @ w1l[...] + xp[...] @ w1r[...] + b1[...]
  h = jnp.maximum(h, 0.0)
  h_out[...] = h
  y2_out[...] = h @ w2l[...]
  z2_out[...] = h @ w2r[...] + b2[...]


_tc1 = pl.pallas_call(
    _tc1_body,
    grid=(GRID,),
    in_specs=[
        pl.BlockSpec((NC, BLK, D_IN), lambda i: (0, i, 0)),
        pl.BlockSpec((BLK, NC), lambda i: (i, 0)),
        pl.BlockSpec((BLK, D_IN), lambda i: (i, 0)),
        pl.BlockSpec((D_IN, D_HID), lambda i: (0, 0)),
        pl.BlockSpec((D_IN, D_HID), lambda i: (0, 0)),
        pl.BlockSpec((1, D_HID), lambda i: (0, 0)),
        pl.BlockSpec((D_HID, N_CLS), lambda i: (0, 0)),
        pl.BlockSpec((D_HID, N_CLS), lambda i: (0, 0)),
        pl.BlockSpec((1, N_CLS), lambda i: (0, 0)),
    ],
    out_specs=[
        pl.BlockSpec((BLK, D_HID), lambda i: (i, 0)),
        pl.BlockSpec((BLK, N_CLS), lambda i: (i, 0)),
        pl.BlockSpec((BLK, N_CLS), lambda i: (i, 0)),
    ],
    out_shape=[
        jax.ShapeDtypeStruct((NP, D_HID), jnp.float32),
        jax.ShapeDtypeStruct((NP, N_CLS), jnp.float32),
        jax.ShapeDtypeStruct((NP, N_CLS), jnp.float32),
    ],
)


def _tc2_body(agg2p, cnt_t, z2, out):
  s = agg2p[0] + agg2p[1]
  cnt = cnt_t[:, 0] + cnt_t[:, 1]
  inv = 1.0 / jnp.maximum(cnt, 1.0)
  out[...] = (s * inv[:, None] + z2[...])[:N_NODES]


_tc2 = pl.pallas_call(
    _tc2_body,
    in_specs=[
        pl.BlockSpec((NC, NP, N_CLS), lambda: (0, 0, 0)),
        pl.BlockSpec((NP, NC), lambda: (0, 0)),
        pl.BlockSpec((NP, N_CLS), lambda: (0, 0)),
    ],
    out_specs=pl.BlockSpec((N_NODES, N_CLS), lambda: (0, 0)),
    out_shape=jax.ShapeDtypeStruct((N_NODES, N_CLS), jnp.float32),
)


def kernel(x, edge_index, W1_l, W1_r, b1, W2_l, W2_r, b2):
  src = edge_index[0].astype(jnp.int32)
  dst = edge_index[1].astype(jnp.int32)
  pad = EFLAT - N_EDGES
  srcp = jnp.concatenate([src, jnp.zeros((pad,), jnp.int32)])
  dstp = jnp.concatenate([dst, jnp.full((pad,), NP - 1, jnp.int32)])

  aggp, cntp = _agg_l1(x, srcp.reshape(M1, CHUNK1), dstp.reshape(M1, CHUNK1))
  cnt_t = cntp.T
  xp = jnp.pad(x, ((0, NP - N_NODES), (0, 0)))

  h, y2, z2 = _tc1(aggp, cnt_t, xp, W1_l, W1_r, b1.reshape(1, -1),
                   W2_l, W2_r, b2.reshape(1, -1))

  (agg2p,) = _agg_l2(y2, srcp.reshape(M2, CHUNK2), dstp.reshape(M2, CHUNK2))
  return _tc2(agg2p, cnt_t, z2)
